# Initial kernel scaffold; baseline (speedup 1.0000x reference)
import jax
import jax.numpy as jnp
from jax.experimental import pallas as pl


def kernel(hidden_states, routing_mask, Wg0, Wu0, Wd0, Wg1, Wu1, Wd1):
    raise NotImplementedError("write your pallas kernel here")


# speedup vs baseline: 1.0526x; 1.0526x over previous
import functools

import jax
import jax.numpy as jnp
from jax.experimental import pallas as pl
from jax.experimental.pallas import tpu as pltpu


def _mlp_block_kernel(eref, x_ref, wg_hbm, wu_hbm, wd_hbm, o_ref,
                      wg_v, wu_v, wd_v, sem, *, n_f_chunks):
    t = pl.program_id(0)
    e = eref[t]
    prev_e = eref[jnp.maximum(t - 1, 0)]

    @pl.when(jnp.logical_or(t == 0, e != prev_e))
    def _load_weights():
        cps = [pltpu.make_async_copy(hbm.at[e], v, sem)
               for hbm, v in ((wg_hbm, wg_v), (wu_hbm, wu_v), (wd_hbm, wd_v))]
        for cp in cps:
            cp.start()
        for cp in cps:
            cp.wait()

    x = x_ref[...]
    f = wg_v.shape[1]
    fc = f // n_f_chunks
    acc = jnp.zeros(o_ref.shape, jnp.float32)
    for i in range(n_f_chunks):
        g = jnp.dot(x, wg_v[:, i * fc:(i + 1) * fc],
                    preferred_element_type=jnp.float32)
        u = jnp.dot(x, wu_v[:, i * fc:(i + 1) * fc],
                    preferred_element_type=jnp.float32)
        h = g * jax.lax.logistic(g) * u
        acc = acc + jnp.dot(h, wd_v[i * fc:(i + 1) * fc, :],
                            preferred_element_type=jnp.float32)
    o_ref[...] = acc


def _routed_mlp(x_disp, blk_expert, Wg, Wu, Wd, tb):
    tcap, d = x_disp.shape
    f = Wg.shape[2]
    nb = tcap // tb
    grid_spec = pltpu.PrefetchScalarGridSpec(
        num_scalar_prefetch=1,
        grid=(nb,),
        in_specs=[
            pl.BlockSpec((tb, d), lambda t, e: (t, 0)),
            pl.BlockSpec(memory_space=pl.ANY),
            pl.BlockSpec(memory_space=pl.ANY),
            pl.BlockSpec(memory_space=pl.ANY),
        ],
        out_specs=pl.BlockSpec((tb, d), lambda t, e: (t, 0)),
        scratch_shapes=[
            pltpu.VMEM((d, f), jnp.float32),
            pltpu.VMEM((d, f), jnp.float32),
            pltpu.VMEM((f, d), jnp.float32),
            pltpu.SemaphoreType.DMA,
        ],
    )
    return pl.pallas_call(
        functools.partial(_mlp_block_kernel, n_f_chunks=4),
        grid_spec=grid_spec,
        out_shape=jax.ShapeDtypeStruct((tcap, d), jnp.float32),
    )(blk_expert, x_disp, Wg, Wu, Wd)


def kernel(hidden_states, routing_mask, Wg0, Wu0, Wd0, Wg1, Wu1, Wd1):
    b, s, d = hidden_states.shape
    t = b * s
    tb = min(512, t)
    x2d = hidden_states.reshape(t, d)
    mask = routing_mask.reshape(t)

    is1 = (mask == 1).astype(jnp.int32)
    c0 = t - jnp.sum(is1)
    cap0 = ((c0 + tb - 1) // tb) * tb
    pos = jnp.where(
        is1 == 0,
        jnp.cumsum(1 - is1) - 1,
        cap0 + jnp.cumsum(is1) - 1,
    ).astype(jnp.int32)

    tcap = t + tb
    nb = tcap // tb
    blk_expert = (jnp.arange(nb, dtype=jnp.int32) * tb >= cap0).astype(jnp.int32)

    x_disp = jnp.zeros((tcap, d), jnp.float32).at[pos].set(x2d)

    Wg = jnp.stack([Wg0, Wg1])
    Wu = jnp.stack([Wu0, Wu1])
    Wd = jnp.stack([Wd0, Wd1])
    y_disp = _routed_mlp(x_disp, blk_expert, Wg, Wu, Wd, tb)

    return y_disp[pos].reshape(b, s, d)


# speedup vs baseline: 1.3475x; 1.2801x over previous
import functools

import jax
import jax.numpy as jnp
from jax import lax
from jax.experimental import pallas as pl
from jax.experimental.pallas import tpu as pltpu
from jax.experimental.pallas import tpu_sc as plsc

TB = 512
_L = 16
_NW = 32
_RCH = 128


def _dispatch_kernel(mask_hbm, x_hbm, xd_hbm, pos_hbm, blk_hbm,
                     mask_v, rows_v, idx2, blk_v, sem, *, t, d, nblk):
    nc = 2
    wid = lax.axis_index("s") * nc + lax.axis_index("c")
    ct = t // _NW
    base = wid * ct
    nv = t // _L
    myfirst = wid * (ct // _L)

    pltpu.sync_copy(mask_hbm, mask_v)

    def scan_body(j, carry):
        tot, bef = carry
        m = mask_v[pl.ds(j * _L, _L)]
        z = 1 - m
        pred = jnp.where(j < myfirst, 1, 0)
        return tot + z, bef + z * pred

    zero = jnp.zeros((_L,), jnp.int32)
    tot, bef = lax.fori_loop(0, nv, scan_body, (zero, zero))
    c0 = plsc.cumsum(tot)[_L - 1]
    cap0 = ((c0 + TB - 1) // TB) * TB
    off0 = plsc.cumsum(bef)[_L - 1]
    off1 = base - off0

    for g in range(ct // _L):
        m = mask_v[pl.ds(base + g * _L, _L)]
        z = 1 - m
        cs0 = plsc.cumsum(z)
        cs1 = plsc.cumsum(m)
        pos_g = (off0 + cs0 - 1) * z + (cap0 + off1 + cs1 - 1) * m
        ch = (g * _L) // _RCH
        idx2[ch, pl.ds((g * _L) % _RCH, _L)] = pos_g
        s0 = cs0[_L - 1]
        off0 = off0 + s0
        off1 = off1 + (_L - s0)

    for ch in range(ct // _RCH):
        pltpu.sync_copy(idx2.at[ch], pos_hbm.at[pl.ds(base + ch * _RCH, _RCH)])
        pltpu.sync_copy(x_hbm.at[pl.ds(base + ch * _RCH, _RCH)], rows_v)
        pltpu.async_copy(rows_v, xd_hbm.at[idx2.at[ch]], sem).wait()

    @pl.when(wid == 0)
    def _write_blk():
        for g in range(nblk // _L):
            bidx = lax.iota(jnp.int32, _L) + g * _L
            diff = bidx * TB - cap0
            blk_v[pl.ds(g * _L, _L)] = 1 - lax.shift_right_logical(diff, 31)
        pltpu.sync_copy(blk_v, blk_hbm)


def _combine_kernel(yd_hbm, pos_hbm, out_hbm, rows_v, idx2, sem, *, t, d):
    nc = 2
    wid = lax.axis_index("s") * nc + lax.axis_index("c")
    ct = t // _NW
    base = wid * ct
    for ch in range(ct // _RCH):
        pltpu.sync_copy(pos_hbm.at[pl.ds(base + ch * _RCH, _RCH)], idx2.at[ch])
        pltpu.async_copy(yd_hbm.at[idx2.at[ch]], rows_v, sem).wait()
        pltpu.sync_copy(rows_v, out_hbm.at[pl.ds(base + ch * _RCH, _RCH)])


def _mlp_block_kernel(eref, x_ref, wg0_hbm, wu0_hbm, wd0_hbm,
                      wg1_hbm, wu1_hbm, wd1_hbm, o_ref,
                      wg_v, wu_v, wd_v, sem, *, n_f_chunks):
    bt = pl.program_id(0)
    e = eref[bt]
    prev_e = eref[jnp.maximum(bt - 1, 0)]

    @pl.when(jnp.logical_or(bt == 0, e != prev_e))
    def _load_weights():
        @pl.when(e == 0)
        def _():
            cps = [pltpu.make_async_copy(hbm, v, sem)
                   for hbm, v in ((wg0_hbm, wg_v), (wu0_hbm, wu_v),
                                  (wd0_hbm, wd_v))]
            for cp in cps:
                cp.start()
            for cp in cps:
                cp.wait()

        @pl.when(e != 0)
        def _():
            cps = [pltpu.make_async_copy(hbm, v, sem)
                   for hbm, v in ((wg1_hbm, wg_v), (wu1_hbm, wu_v),
                                  (wd1_hbm, wd_v))]
            for cp in cps:
                cp.start()
            for cp in cps:
                cp.wait()

    x = x_ref[...]
    f = wg_v.shape[1]
    fc = f // n_f_chunks
    acc = jnp.zeros(o_ref.shape, jnp.float32)
    for i in range(n_f_chunks):
        g = jnp.dot(x, wg_v[:, i * fc:(i + 1) * fc],
                    preferred_element_type=jnp.float32)
        u = jnp.dot(x, wu_v[:, i * fc:(i + 1) * fc],
                    preferred_element_type=jnp.float32)
        h = g * lax.logistic(g) * u
        acc = acc + jnp.dot(h, wd_v[i * fc:(i + 1) * fc, :],
                            preferred_element_type=jnp.float32)
    o_ref[...] = acc


def _routed_mlp(x_disp, blk_expert, Wg0, Wu0, Wd0, Wg1, Wu1, Wd1):
    tcap, d = x_disp.shape
    f = Wg0.shape[1]
    nb = tcap // TB
    grid_spec = pltpu.PrefetchScalarGridSpec(
        num_scalar_prefetch=1,
        grid=(nb,),
        in_specs=[
            pl.BlockSpec((TB, d), lambda bt, e: (bt, 0)),
        ] + [pl.BlockSpec(memory_space=pl.ANY)] * 6,
        out_specs=pl.BlockSpec((TB, d), lambda bt, e: (bt, 0)),
        scratch_shapes=[
            pltpu.VMEM((d, f), jnp.float32),
            pltpu.VMEM((d, f), jnp.float32),
            pltpu.VMEM((f, d), jnp.float32),
            pltpu.SemaphoreType.DMA,
        ],
    )
    return pl.pallas_call(
        functools.partial(_mlp_block_kernel, n_f_chunks=4),
        grid_spec=grid_spec,
        out_shape=jax.ShapeDtypeStruct((tcap, d), jnp.float32),
    )(blk_expert, x_disp, Wg0, Wu0, Wd0, Wg1, Wu1, Wd1)


def kernel(hidden_states, routing_mask, Wg0, Wu0, Wd0, Wg1, Wu1, Wd1):
    b, s, d = hidden_states.shape
    t = b * s
    x2d = hidden_states.reshape(t, d)
    mask = routing_mask.reshape(t)

    tcap = t + TB
    nb = tcap // TB
    nblk_pad = ((nb + _L - 1) // _L) * _L

    mesh = plsc.VectorSubcoreMesh(core_axis_name="c", subcore_axis_name="s")
    dispatch = pl.kernel(
        functools.partial(_dispatch_kernel, t=t, d=d, nblk=nblk_pad),
        out_type=(
            jax.ShapeDtypeStruct((tcap, d), jnp.float32),
            jax.ShapeDtypeStruct((t,), jnp.int32),
            jax.ShapeDtypeStruct((nblk_pad,), jnp.int32),
        ),
        mesh=mesh,
        compiler_params=pltpu.CompilerParams(needs_layout_passes=False),
        scratch_types=[
            pltpu.VMEM((t,), jnp.int32),
            pltpu.VMEM((_RCH, d), jnp.float32),
            pltpu.VMEM((t // _NW // _RCH, _RCH), jnp.int32),
            pltpu.VMEM((nblk_pad,), jnp.int32),
            pltpu.SemaphoreType.DMA,
        ],
    )
    x_disp, pos, blk = dispatch(mask, x2d)

    y_disp = _routed_mlp(x_disp, blk[:nb], Wg0, Wu0, Wd0, Wg1, Wu1, Wd1)

    combine = pl.kernel(
        functools.partial(_combine_kernel, t=t, d=d),
        out_type=jax.ShapeDtypeStruct((t, d), jnp.float32),
        mesh=mesh,
        scratch_types=[
            pltpu.VMEM((_RCH, d), jnp.float32),
            pltpu.VMEM((t // _NW // _RCH, _RCH), jnp.int32),
            pltpu.SemaphoreType.DMA,
        ],
    )
    out2d = combine(y_disp, pos)
    return out2d.reshape(b, s, d)
